# baseline (speedup 1.0000x reference)
import jax
import jax.numpy as jnp
from jax import lax
from jax.experimental import pallas as pl
from jax.experimental.pallas import tpu as pltpu
from jax.experimental.pallas import tpu_sc as plsc

VOCAB = 1000000
EMB = 64
BATCH = 16384
HIST = 200
SCALE = EMB ** (-0.5)

NC = 2
NS = 16
NW = NC * NS
TILE = 128
NT = BATCH // TILE
TPW = NT // NW


def _sc_body(x_hbm, table_hbm, out_hbm, idx_v, rows_v, tv_v, gs, ss, isem):
    c = lax.axis_index("c")
    s = lax.axis_index("s")
    wid = s * NC + c

    lanes = lax.iota(jnp.int32, 16)
    zeros16 = jnp.zeros((16,), jnp.int32)

    def fire_idx(k):
        pltpu.async_copy(
            x_hbm.at[:, pl.ds((wid * TPW + k) * TILE, TILE)],
            idx_v.at[k & 1], isem.at[k & 1])

    def wait_idx(k):
        pltpu.make_async_copy(
            x_hbm.at[:, pl.ds((wid * TPW + k) * TILE, TILE)],
            idx_v.at[k & 1], isem.at[k & 1]).wait()

    def fire_gather(k, h, hb):
        pltpu.async_copy(
            table_hbm.at[idx_v.at[k & 1, h, :]],
            rows_v.at[pl.ds(hb * TILE, TILE)], gs.at[hb])

    def wait_gather(k, h, hb):
        pltpu.make_async_copy(
            table_hbm.at[idx_v.at[k & 1, h, :]],
            rows_v.at[pl.ds(hb * TILE, TILE)], gs.at[hb]).wait()

    def fire_scatter(k, h, p):
        for tr in range(8):
            pltpu.async_copy(
                tv_v.at[p, pl.ds(tr * 1024, 1024)],
                out_hbm.at[h, tr, wid * TPW + k, :], ss.at[p])

    def wait_scatter(k, h, p):
        for tr in range(8):
            pltpu.make_async_copy(
                tv_v.at[p, pl.ds(tr * 1024, 1024)],
                out_hbm.at[h, tr, wid * TPW + k, :], ss.at[p]).wait()

    def transpose_scale(hb, p):
        b0 = hb * TILE
        p_v = zeros16 + p
        row_vs = [b0 + bb * 16 + lanes for bb in range(8)]

        @pl.loop(0, 16)
        def _t(t):
            perm = (lanes + t) & 15
            for eb in range(4):
                e_v = eb * 16 + perm
                posbase = e_v * 128 + lanes
                for bb in range(8):
                    v = plsc.load_gather(rows_v, [row_vs[bb], e_v])
                    plsc.store_scatter(
                        tv_v, [p_v, posbase + bb * 16], v * SCALE)

    fire_idx(0)
    wait_idx(0)
    fire_gather(0, 0, 0)
    fire_gather(0, 1, 1)

    @pl.loop(0, TPW)
    def _kloop(k):
        @pl.loop(0, HIST)
        def _hloop(h):
            hb = h & 3
            p = h & 1

            @pl.when(jnp.logical_and(h == 0, k + 1 < TPW))
            def _():
                fire_idx(k + 1)

            @pl.when(jnp.logical_and(h == HIST - 2, k + 1 < TPW))
            def _():
                wait_idx(k + 1)

            @pl.when(h < HIST - 2)
            def _():
                fire_gather(k, h + 2, (h + 2) & 3)

            @pl.when(jnp.logical_and(h >= HIST - 2, k + 1 < TPW))
            def _():
                fire_gather(k + 1, h - (HIST - 2), (h + 2) & 3)

            wait_gather(k, h, hb)

            @pl.when(jnp.logical_or(h >= 2, k > 0))
            def _():
                wait_scatter(k, h, p)

            transpose_scale(hb, p)
            fire_scatter(k, h, p)

    wait_scatter(TPW - 1, HIST - 2, 0)
    wait_scatter(TPW - 1, HIST - 1, 1)


@jax.jit
def _run(x_t, table):
    mesh = plsc.VectorSubcoreMesh(core_axis_name="c", subcore_axis_name="s",
                                  num_cores=NC, num_subcores=NS)
    f = pl.kernel(
        _sc_body,
        out_type=jax.ShapeDtypeStruct((HIST, 8, NT, 1024), jnp.float32),
        mesh=mesh,
        compiler_params=pltpu.CompilerParams(use_tc_tiling_on_sc=False,
                                             needs_layout_passes=False),
        scratch_types=[
            pltpu.VMEM((2, HIST, TILE), jnp.int32),
            pltpu.VMEM((4 * TILE, EMB), jnp.float32),
            pltpu.VMEM((2, 8192), jnp.float32),
            pltpu.SemaphoreType.DMA((4,)),
            pltpu.SemaphoreType.DMA((2,)),
            pltpu.SemaphoreType.DMA((2,)),
        ],
    )
    return f(x_t, table)


def kernel(x, table):
    x_t = jnp.transpose(x.astype(jnp.int32))
    out = _run(x_t, table)
    out = out.reshape(HIST, 8, NT, 8, 128)
    out = out.transpose(2, 4, 0, 1, 3)
    return out.reshape(BATCH, HIST, EMB)


# speedup vs baseline: 1.4092x; 1.4092x over previous
import jax
import jax.numpy as jnp
from jax import lax
from jax.experimental import pallas as pl
from jax.experimental.pallas import tpu as pltpu
from jax.experimental.pallas import tpu_sc as plsc

VOCAB = 1000000
EMB = 64
EMBP = 128
BATCH = 16384
HIST = 200
SCALE = EMB ** (-0.5)

NC = 2
NS = 16
NW = NC * NS

B = BATCH * HIST
ROWS_PER_W = BATCH // NW
CLOOK = HIST
NCHUNK = ROWS_PER_W
NBUF = 4
G_SPLIT = (128, HIST - 128)


def _pieces(b):
    off = b * CLOOK
    return [(off, G_SPLIT[0]), (off + G_SPLIT[0], G_SPLIT[1])]


def _sc_body(x_hbm, table_hbm, out_hbm, idx_v, rows_v,
             gs0, gs1, gs2, gs3, ss0, ss1, ss2, ss3):
    gs = [gs0, gs1, gs2, gs3]
    ss = [ss0, ss1, ss2, ss3]
    c = lax.axis_index("c")
    s = lax.axis_index("s")
    wid = s * NC + c
    row_base = wid * ROWS_PER_W

    def stage_and_fire_gather(g, b):
        pltpu.sync_copy(
            x_hbm.at[pl.ds((row_base + g) * HIST, CLOOK)],
            idx_v.at[pl.ds(b * CLOOK, CLOOK)])
        for off, ln in _pieces(b):
            pltpu.async_copy(
                table_hbm.at[idx_v.at[pl.ds(off, ln)]],
                rows_v.at[pl.ds(off, ln)],
                gs[b])

    def wait_gather(b):
        for off, ln in _pieces(b):
            pltpu.make_async_copy(
                table_hbm.at[idx_v.at[pl.ds(off, ln)]],
                rows_v.at[pl.ds(off, ln)],
                gs[b]).wait()

    def fire_scatter(g, b):
        pltpu.async_copy(rows_v.at[pl.ds(b * CLOOK, CLOOK)],
                         out_hbm.at[row_base + g, :, pl.ds(0, EMB)], ss[b])

    def wait_scatter(g, b):
        pltpu.make_async_copy(rows_v.at[pl.ds(b * CLOOK, CLOOK)],
                              out_hbm.at[row_base + g, :, pl.ds(0, EMB)],
                              ss[b]).wait()

    stage_and_fire_gather(0, 0)
    stage_and_fire_gather(1, 1)

    @pl.loop(0, NCHUNK // NBUF)
    def _outer(go):
        for b in range(NBUF):
            g = go * NBUF + b
            b2 = (b + 2) % NBUF

            @pl.when(g >= 2)
            def _():
                wait_scatter(g - 2, b2)

            @pl.when(g + 2 < NCHUNK)
            def _():
                stage_and_fire_gather(g + 2, b2)

            wait_gather(b)

            @pl.loop(0, CLOOK, unroll=8)
            def _scale(r):
                for jj in range(EMB // 16):
                    sl = pl.ds(jj * 16, 16)
                    rows_v[b * CLOOK + r, sl] = rows_v[b * CLOOK + r, sl] * SCALE

            fire_scatter(g, b)

    wait_scatter(NCHUNK - 2, (NCHUNK - 2) % NBUF)
    wait_scatter(NCHUNK - 1, (NCHUNK - 1) % NBUF)


@jax.jit
def _run(x_flat, table):
    mesh = plsc.VectorSubcoreMesh(core_axis_name="c", subcore_axis_name="s",
                                  num_cores=NC, num_subcores=NS)
    f = pl.kernel(
        _sc_body,
        out_type=jax.ShapeDtypeStruct((BATCH, HIST, EMBP), jnp.float32),
        mesh=mesh,
        compiler_params=pltpu.CompilerParams(use_tc_tiling_on_sc=False),
        scratch_types=[
            pltpu.VMEM((NBUF * CLOOK,), jnp.int32),
            pltpu.VMEM((NBUF * CLOOK, EMB), jnp.float32),
        ] + [pltpu.SemaphoreType.DMA] * (2 * NBUF),
    )
    return f(x_flat, table)


def kernel(x, table):
    x_flat = x.astype(jnp.int32).reshape(B)
    out = _run(x_flat, table)
    return out[:, :, :EMB]
